# baseline (speedup 1.0000x reference)
import functools

import jax
import jax.numpy as jnp
from jax import lax
from jax.experimental import pallas as pl
from jax.experimental.pallas import tpu as pltpu
from jax.experimental.pallas import tpu_sc as plsc

N_CH = 144
Z = 1024
DIM_OUT = 10440
NNZ = 82944

NC = 2
NS = 16
ZB = 128
NCHUNK = Z // ZB
BATCH = 64
NB = 81
SB = 3
NSB = NB // SB
ROWS_TEC = 656
R_PAD = NS * ROWS_TEC


def _sc_spmm(ftc, meta, vals_a):
    mesh = plsc.VectorSubcoreMesh(core_axis_name="c", subcore_axis_name="s")

    @functools.partial(
        pl.kernel,
        out_type=jax.ShapeDtypeStruct((NCHUNK, R_PAD, ZB), jnp.float32),
        mesh=mesh,
        scratch_types=[
            pltpu.VMEM((N_CH, ZB), jnp.float32),
            pltpu.VMEM((2, SB, 2, BATCH), jnp.int32),
            pltpu.VMEM((2, SB, BATCH), jnp.float32),
            pltpu.VMEM((BATCH, ZB), jnp.float32),
            pltpu.VMEM((BATCH,), jnp.int32),
            pltpu.VMEM_SHARED((R_PAD, ZB), jnp.float32),
            pltpu.SemaphoreType.DMA,
            pltpu.SemaphoreType.DMA,
        ],
    )
    def body(ftc_hbm, m_hbm, v_hbm, out_hbm, ft_v, m_v, v_v, c_v, idx_v, acc_sh, sem0, sem1):
        c = lax.axis_index("c")
        s = lax.axis_index("s")
        sems = (sem0, sem1)

        zero16 = jnp.zeros((16,), jnp.float32)

        def process_batch(buf, k):
            def group_body(g, carry2):
                pv = m_v[buf, k, 0, pl.ds(g * 16, 16)]
                iv = lax.shift_right_logical(pv, 8)
                jv = lax.bitwise_and(pv, 255)
                vv = v_v[buf, k, pl.ds(g * 16, 16)]
                for l in range(16):
                    ik = iv[l]
                    jk = jv[l]
                    vk = vv[l]
                    for u in range(ZB // 16):
                        av = ft_v[ik, pl.ds(u * 16, 16)]
                        bv = ft_v[jk, pl.ds(u * 16, 16)]
                        c_v[g * 16 + l, pl.ds(u * 16, 16)] = av * bv * vk
                return carry2

            lax.fori_loop(0, BATCH // 16, group_body, 0)
            for g in range(BATCH // 16):
                idx_v[pl.ds(g * 16, 16)] = m_v[buf, k, 1, pl.ds(g * 16, 16)]
            pltpu.sync_copy(c_v, acc_sh.at[idx_v], add=True)

        def chunk_body(ci, carry):
            zc = ci * NC + c

            def zero_body(b, carry0):
                for u in range(ZB // 16):
                    c_v[b, pl.ds(u * 16, 16)] = zero16
                return carry0

            lax.fori_loop(0, BATCH, zero_body, 0)
            for t in range(ROWS_TEC // BATCH):
                pltpu.sync_copy(
                    c_v, acc_sh.at[pl.ds(s * ROWS_TEC + t * BATCH, BATCH)])
            rem = ROWS_TEC - (ROWS_TEC // BATCH) * BATCH
            if rem:
                pltpu.sync_copy(
                    c_v.at[pl.ds(0, rem)],
                    acc_sh.at[pl.ds(s * ROWS_TEC + (ROWS_TEC // BATCH) * BATCH,
                                    rem)])

            pltpu.sync_copy(ftc_hbm.at[zc], ft_v)
            plsc.subcore_barrier()

            pltpu.async_copy(m_hbm.at[s, 0], m_v.at[0], sem0)
            pltpu.async_copy(v_hbm.at[s, 0], v_v.at[0], sem0)

            def pair_body(h, carry1):
                sb0 = h * 2
                for par in range(2):
                    sbi = sb0 + par
                    pltpu.make_async_copy(
                        m_hbm.at[s, sbi], m_v.at[par], sems[par]).wait()
                    pltpu.make_async_copy(
                        v_hbm.at[s, sbi], v_v.at[par], sems[par]).wait()
                    pltpu.async_copy(
                        m_hbm.at[s, sbi + 1], m_v.at[1 - par], sems[1 - par])
                    pltpu.async_copy(
                        v_hbm.at[s, sbi + 1], v_v.at[1 - par], sems[1 - par])
                    for k in range(SB):
                        process_batch(par, k)
                return carry1

            lax.fori_loop(0, (NSB - 1) // 2, pair_body, 0)
            pltpu.make_async_copy(
                m_hbm.at[s, NSB - 1], m_v.at[0], sem0).wait()
            pltpu.make_async_copy(
                v_hbm.at[s, NSB - 1], v_v.at[0], sem0).wait()
            for k in range(SB):
                process_batch(0, k)

            plsc.subcore_barrier()
            pltpu.sync_copy(acc_sh.at[pl.ds(s * ROWS_TEC, ROWS_TEC)],
                            out_hbm.at[zc, pl.ds(s * ROWS_TEC, ROWS_TEC)])
            return carry

        lax.fori_loop(0, NCHUNK // NC, chunk_body, 0)

    return body(ftc, meta, vals_a)


def _transpose_tc(out_c):
    def tbody(x_ref, o_ref):
        x = x_ref[0]
        o_ref[...] = x[:DIM_OUT, :].T

    return pl.pallas_call(
        tbody,
        grid=(NCHUNK,),
        in_specs=[pl.BlockSpec((1, R_PAD, ZB), lambda zc: (zc, 0, 0))],
        out_specs=pl.BlockSpec((ZB, DIM_OUT), lambda zc: (zc, 0)),
        out_shape=jax.ShapeDtypeStruct((Z, DIM_OUT), jnp.float32),
    )(out_c)


def kernel(features, mix_rows, mix_cols, mix_vals):
    f = features.reshape(-1, N_CH)
    ftc = f.T.reshape(N_CH, NCHUNK, ZB).transpose(1, 0, 2)

    cols = mix_cols.astype(jnp.int32)
    i_idx = cols // N_CH
    j_idx = cols - i_idx * N_CH
    packed = (i_idx << 8) | j_idx
    rows = mix_rows.astype(jnp.int32)
    meta = jnp.stack(
        [packed.reshape(NS, NSB, SB, BATCH),
         rows.reshape(NS, NSB, SB, BATCH)], axis=3)
    vals_a = mix_vals.astype(jnp.float32).reshape(NS, NSB, SB, BATCH)

    out_c = _sc_spmm(ftc, meta, vals_a)
    return _transpose_tc(out_c)
